# Initial kernel scaffold; baseline (speedup 1.0000x reference)
import functools

import jax
import jax.numpy as jnp
from jax import lax
from jax.experimental import pallas as pl
from jax.experimental.pallas import tpu as pltpu
from jax.experimental.pallas import tpu_sc as plsc

KNN = 3
B = 4
N = 16384
M = 4096
D = 256

TN = 256
NT = N // TN

NC = 2
NS = 16
LANES = 16
NW = NC * NS
TOT = B * N
PW = TOT // NW
C = 32
NCHUNK = PW // C


def _topk_body(xyz_ref, sxyz_ref, mask_ref, gidx_ref, wb_ref):
    b = pl.program_id(0)
    x = xyz_ref[0]
    s = sxyz_ref[0]
    mask = mask_ref[0]

    d = jnp.zeros((TN, M), jnp.float32)
    for c in range(3):
        diff = x[:, c:c + 1] - s[c:c + 1, :]
        d = d + diff * diff

    iota = lax.broadcasted_iota(jnp.int32, (TN, M), 1)
    inf = jnp.float32(jnp.inf)

    def extract(dcur):
        mval = jnp.min(dcur, axis=1, keepdims=True)
        aidx = jnp.min(jnp.where(dcur == mval, iota, M), axis=1, keepdims=True)
        return mval, aidx

    m1, a1 = extract(d)
    d = jnp.where(iota == a1, inf, d)
    m2, a2 = extract(d)
    d = jnp.where(iota == a2, inf, d)
    m3, a3 = extract(d)

    def weight(mv):
        v = jnp.maximum(mv, jnp.float32(1e-10))
        return 1.0 / (v * v + jnp.float32(1e-10))

    w1 = weight(m1)
    w2 = weight(m2)
    w3 = weight(m3)
    wsum = w1 + w2 + w3
    w1 = w1 / wsum * mask
    w2 = w2 / wsum * mask
    w3 = w3 / wsum * mask

    gidx_ref[0] = jnp.concatenate([a1, a2, a3], axis=1) + b * M
    wb_ref[0] = jnp.concatenate([
        jnp.broadcast_to(w1, (TN, LANES)),
        jnp.broadcast_to(w2, (TN, LANES)),
        jnp.broadcast_to(w3, (TN, LANES)),
    ], axis=1)


def _tc_topk(xyz, sxyz_t, mask_f):
    return pl.pallas_call(
        _topk_body,
        grid=(B, NT),
        in_specs=[
            pl.BlockSpec((1, TN, 3), lambda b, t: (b, t, 0)),
            pl.BlockSpec((1, 3, M), lambda b, t: (b, 0, 0)),
            pl.BlockSpec((1, TN, 1), lambda b, t: (b, t, 0)),
        ],
        out_specs=[
            pl.BlockSpec((1, TN, KNN), lambda b, t: (b, t, 0)),
            pl.BlockSpec((1, TN, KNN * LANES), lambda b, t: (b, t, 0)),
        ],
        out_shape=[
            jax.ShapeDtypeStruct((B, N, KNN), jnp.int32),
            jax.ShapeDtypeStruct((B, N, KNN * LANES), jnp.float32),
        ],
    )(xyz, sxyz_t, mask_f)


def _sc_interp(table, gidx, wb):
    mesh = plsc.VectorSubcoreMesh(core_axis_name="c", subcore_axis_name="s")

    @functools.partial(
        pl.kernel,
        out_type=jax.ShapeDtypeStruct((TOT, D), jnp.float32),
        mesh=mesh,
        scratch_types=[
            pltpu.VMEM((KNN * C,), jnp.int32),
            pltpu.VMEM((KNN * C, LANES), jnp.float32),
            pltpu.VMEM((KNN * C, D), jnp.float32),
            pltpu.VMEM((C, D), jnp.float32),
            pltpu.SemaphoreType.DMA,
        ],
    )
    def sck(table_hbm, gidx_hbm, wb_hbm, out_hbm, idx_v, w_v, rows_v, out_v, sem):
        wid = lax.axis_index("s") * NC + lax.axis_index("c")

        @pl.loop(0, NCHUNK)
        def _(ch):
            base = wid * PW + ch * C
            ibase = base * KNN
            pltpu.sync_copy(gidx_hbm.at[pl.ds(ibase, KNN * C)], idx_v)
            pltpu.async_copy(table_hbm.at[idx_v], rows_v, sem).wait()
            pltpu.sync_copy(wb_hbm.at[pl.ds(ibase, KNN * C)], w_v)

            @pl.loop(0, C)
            def _(r):
                w0 = w_v[KNN * r]
                w1 = w_v[KNN * r + 1]
                w2 = w_v[KNN * r + 2]
                for c in range(D // LANES):
                    sl = pl.ds(c * LANES, LANES)
                    out_v[r, sl] = (rows_v[KNN * r, sl] * w0 +
                                    rows_v[KNN * r + 1, sl] * w1 +
                                    rows_v[KNN * r + 2, sl] * w2)

            pltpu.sync_copy(out_v, out_hbm.at[pl.ds(base, C)])

    return sck(table, gidx, wb)


@jax.jit
def kernel(xyz, sampled_xyz, features, sampled_features, masks):
    del features
    sxyz_t = sampled_xyz.transpose(0, 2, 1)
    mask_f = masks.astype(jnp.float32).reshape(B, N, 1)
    gidx, wb = _tc_topk(xyz, sxyz_t, mask_f)
    table = sampled_features.reshape(B * M, D)
    gidx_flat = gidx.reshape(TOT * KNN)
    wb_flat = wb.reshape(TOT * KNN, LANES)
    out = _sc_interp(table, gidx_flat, wb_flat)
    return out.reshape(B, N, D)


# speedup vs baseline: 10.8632x; 10.8632x over previous
import functools

import jax
import jax.numpy as jnp
from jax import lax
from jax.experimental import pallas as pl
from jax.experimental.pallas import tpu as pltpu
from jax.experimental.pallas import tpu_sc as plsc

KNN = 3
B = 4
N = 16384
M = 4096
D = 256

TN = 256
NT = N // TN

NC = 2
NS = 16
LANES = 16
NW = NC * NS
TOT = B * N
PW = TOT // NW
C = 32
NCHUNK = PW // C


def _topk_body(xyz_ref, sxyz_ref, mask_ref, gidx_ref, wb_ref):
    b = pl.program_id(0)
    x = xyz_ref[0]
    s = sxyz_ref[0]
    mask = mask_ref[0]

    x16 = x.astype(jnp.bfloat16).astype(jnp.float32)
    s16 = s.astype(jnp.bfloat16).astype(jnp.float32)
    a2 = jnp.sum(x * x, axis=1, keepdims=True)
    b2 = jnp.sum(s * s, axis=0, keepdims=True)
    cross = (x16[:, 0:1] * s16[0:1, :] +
             x16[:, 1:2] * s16[1:2, :] +
             x16[:, 2:3] * s16[2:3, :])
    d = a2 + b2 - 2.0 * cross

    iota = lax.broadcasted_iota(jnp.int32, (TN, M), 1)
    inf = jnp.float32(jnp.inf)

    def extract(dcur):
        mval = jnp.min(dcur, axis=1, keepdims=True)
        aidx = jnp.min(jnp.where(dcur == mval, iota, M), axis=1, keepdims=True)
        return mval, aidx

    m1, a1 = extract(d)
    d = jnp.where(iota == a1, inf, d)
    m2, a2 = extract(d)
    d = jnp.where(iota == a2, inf, d)
    m3, a3 = extract(d)

    def weight(mv):
        v = jnp.maximum(mv, jnp.float32(1e-10))
        return 1.0 / (v * v + jnp.float32(1e-10))

    w1 = weight(m1)
    w2 = weight(m2)
    w3 = weight(m3)
    wsum = w1 + w2 + w3
    w1 = w1 / wsum * mask
    w2 = w2 / wsum * mask
    w3 = w3 / wsum * mask

    gidx_ref[0] = jnp.concatenate([a1, a2, a3], axis=1) + b * M
    wb_ref[0] = jnp.concatenate([
        jnp.broadcast_to(w1, (TN, LANES)),
        jnp.broadcast_to(w2, (TN, LANES)),
        jnp.broadcast_to(w3, (TN, LANES)),
    ], axis=1)


def _tc_topk(xyz, sxyz_t, mask_f):
    return pl.pallas_call(
        _topk_body,
        grid=(B, NT),
        in_specs=[
            pl.BlockSpec((1, TN, 3), lambda b, t: (b, t, 0)),
            pl.BlockSpec((1, 3, M), lambda b, t: (b, 0, 0)),
            pl.BlockSpec((1, TN, 1), lambda b, t: (b, t, 0)),
        ],
        out_specs=[
            pl.BlockSpec((1, TN, KNN), lambda b, t: (b, t, 0)),
            pl.BlockSpec((1, TN, KNN * LANES), lambda b, t: (b, t, 0)),
        ],
        out_shape=[
            jax.ShapeDtypeStruct((B, N, KNN), jnp.int32),
            jax.ShapeDtypeStruct((B, N, KNN * LANES), jnp.float32),
        ],
    )(xyz, sxyz_t, mask_f)


def _sc_interp(table, gidx, wb):
    mesh = plsc.VectorSubcoreMesh(core_axis_name="c", subcore_axis_name="s")

    @functools.partial(
        pl.kernel,
        out_type=jax.ShapeDtypeStruct((TOT, D), jnp.float32),
        mesh=mesh,
        scratch_types=[
            pltpu.VMEM((KNN * C,), jnp.int32),
            pltpu.VMEM((KNN * C, LANES), jnp.float32),
            pltpu.VMEM((KNN * C, D), jnp.float32),
            pltpu.VMEM((C, D), jnp.float32),
            pltpu.SemaphoreType.DMA,
        ],
    )
    def sck(table_hbm, gidx_hbm, wb_hbm, out_hbm, idx_v, w_v, rows_v, out_v, sem):
        wid = lax.axis_index("s") * NC + lax.axis_index("c")

        @pl.loop(0, NCHUNK)
        def _(ch):
            base = wid * PW + ch * C
            ibase = base * KNN
            pltpu.sync_copy(gidx_hbm.at[pl.ds(ibase, KNN * C)], idx_v)
            pltpu.async_copy(table_hbm.at[idx_v], rows_v, sem).wait()
            pltpu.sync_copy(wb_hbm.at[pl.ds(ibase, KNN * C)], w_v)

            @pl.loop(0, C)
            def _(r):
                w0 = w_v[KNN * r]
                w1 = w_v[KNN * r + 1]
                w2 = w_v[KNN * r + 2]
                for c in range(D // LANES):
                    sl = pl.ds(c * LANES, LANES)
                    out_v[r, sl] = (rows_v[KNN * r, sl] * w0 +
                                    rows_v[KNN * r + 1, sl] * w1 +
                                    rows_v[KNN * r + 2, sl] * w2)

            pltpu.sync_copy(out_v, out_hbm.at[pl.ds(base, C)])

    return sck(table, gidx, wb)


@jax.jit
def kernel(xyz, sampled_xyz, features, sampled_features, masks):
    del features
    sxyz_t = sampled_xyz.transpose(0, 2, 1)
    mask_f = masks.astype(jnp.float32).reshape(B, N, 1)
    gidx, wb = _tc_topk(xyz, sxyz_t, mask_f)
    table = sampled_features.reshape(B * M, D)
    gidx_flat = gidx.reshape(TOT * KNN)
    wb_flat = wb.reshape(TOT * KNN, LANES)
    out = _sc_interp(table, gidx_flat, wb_flat)
    return out.reshape(B, N, D)


# speedup vs baseline: 14.5967x; 1.3437x over previous
import functools

import jax
import jax.numpy as jnp
from jax import lax
from jax.experimental import pallas as pl
from jax.experimental.pallas import tpu as pltpu
from jax.experimental.pallas import tpu_sc as plsc

KNN = 3
B = 4
N = 16384
M = 4096
D = 256

TN = 256
NT = N // TN

NC = 2
NS = 16
LANES = 16
NW = NC * NS
PW = N // NW
C = 32
NCHUNK = PW // C


def _topk_body(xyz_ref, sxyz_ref, mask_ref, gidx_ref, wb_ref):
    x = xyz_ref[...]
    s = sxyz_ref[...]
    mask = mask_ref[...]

    x16 = x.astype(jnp.bfloat16)
    s16 = s.astype(jnp.bfloat16)
    a2 = jnp.sum(x * x, axis=1, keepdims=True)
    b2 = jnp.sum(s * s, axis=0, keepdims=True)
    cross = lax.dot_general(x16, s16, (((1,), (0,)), ((), ())),
                            preferred_element_type=jnp.float32)
    d = a2 + b2 - 2.0 * cross

    iota = lax.broadcasted_iota(jnp.int32, (TN, M), 1)
    inf = jnp.float32(jnp.inf)

    def extract(dcur):
        mval = jnp.min(dcur, axis=1, keepdims=True)
        aidx = jnp.min(jnp.where(dcur == mval, iota, M), axis=1, keepdims=True)
        return mval, aidx

    m1, a1 = extract(d)
    d = jnp.where(iota == a1, inf, d)
    m2, a2 = extract(d)
    d = jnp.where(iota == a2, inf, d)
    m3, a3 = extract(d)

    def weight(mv):
        v = jnp.maximum(mv, jnp.float32(1e-10))
        return 1.0 / (v * v + jnp.float32(1e-10))

    w1 = weight(m1)
    w2 = weight(m2)
    w3 = weight(m3)
    wsum = w1 + w2 + w3
    w1 = w1 / wsum * mask
    w2 = w2 / wsum * mask
    w3 = w3 / wsum * mask

    gidx_ref[...] = jnp.concatenate([a1, a2, a3], axis=1)
    wb_ref[...] = jnp.concatenate([
        jnp.broadcast_to(w1, (TN, LANES)),
        jnp.broadcast_to(w2, (TN, LANES)),
        jnp.broadcast_to(w3, (TN, LANES)),
    ], axis=1)


def _tc_topk(xyz_b, sxyz_t_b, mask_b):
    return pl.pallas_call(
        _topk_body,
        grid=(NT,),
        in_specs=[
            pl.BlockSpec((TN, 3), lambda t: (t, 0)),
            pl.BlockSpec((3, M), lambda t: (0, 0)),
            pl.BlockSpec((TN, 1), lambda t: (t, 0)),
        ],
        out_specs=[
            pl.BlockSpec((TN, KNN), lambda t: (t, 0)),
            pl.BlockSpec((TN, KNN * LANES), lambda t: (t, 0)),
        ],
        out_shape=[
            jax.ShapeDtypeStruct((N, KNN), jnp.int32),
            jax.ShapeDtypeStruct((N, KNN * LANES), jnp.float32),
        ],
    )(xyz_b, sxyz_t_b, mask_b)


def _sc_interp(table, gidx, wb):
    mesh = plsc.VectorSubcoreMesh(core_axis_name="c", subcore_axis_name="s")

    @functools.partial(
        pl.kernel,
        out_type=jax.ShapeDtypeStruct((N, D), jnp.float32),
        mesh=mesh,
        scratch_types=[
            pltpu.VMEM((KNN * C,), jnp.int32),
            pltpu.VMEM((KNN * C, LANES), jnp.float32),
            pltpu.VMEM((KNN * C, D), jnp.float32),
            pltpu.VMEM((C, D), jnp.float32),
            pltpu.SemaphoreType.DMA,
        ],
    )
    def sck(table_hbm, gidx_hbm, wb_hbm, out_hbm, idx_v, w_v, rows_v, out_v, sem):
        wid = lax.axis_index("s") * NC + lax.axis_index("c")

        @pl.loop(0, NCHUNK)
        def _(ch):
            base = wid * PW + ch * C
            ibase = base * KNN
            pltpu.sync_copy(gidx_hbm.at[pl.ds(ibase, KNN * C)], idx_v)
            pltpu.async_copy(table_hbm.at[idx_v], rows_v, sem).wait()
            pltpu.sync_copy(wb_hbm.at[pl.ds(ibase, KNN * C)], w_v)

            @pl.loop(0, C)
            def _(r):
                w0 = w_v[KNN * r]
                w1 = w_v[KNN * r + 1]
                w2 = w_v[KNN * r + 2]
                for c in range(D // LANES):
                    sl = pl.ds(c * LANES, LANES)
                    out_v[r, sl] = (rows_v[KNN * r, sl] * w0 +
                                    rows_v[KNN * r + 1, sl] * w1 +
                                    rows_v[KNN * r + 2, sl] * w2)

            pltpu.sync_copy(out_v, out_hbm.at[pl.ds(base, C)])

    return sck(table, gidx, wb)


@jax.jit
def kernel(xyz, sampled_xyz, features, sampled_features, masks):
    del features
    sxyz_t = sampled_xyz.transpose(0, 2, 1)
    mask_f = masks.astype(jnp.float32).reshape(B, N, 1)
    outs = []
    for b in range(B):
        gidx, wb = _tc_topk(xyz[b], sxyz_t[b], mask_f[b])
        out = _sc_interp(sampled_features[b],
                         gidx.reshape(N * KNN),
                         wb.reshape(N * KNN, LANES))
        outs.append(out)
    return jnp.stack(outs, axis=0)


# speedup vs baseline: 16.9244x; 1.1595x over previous
import functools

import jax
import jax.numpy as jnp
from jax import lax
from jax.experimental import pallas as pl
from jax.experimental.pallas import tpu as pltpu
from jax.experimental.pallas import tpu_sc as plsc

KNN = 3
B = 4
N = 16384
M = 4096
D = 256

TN = 256
NT = N // TN

NC = 2
NS = 16
LANES = 16
NW = NC * NS
PW = N // NW
C = 32
NCHUNK = PW // C


def _topk_body(xyz_ref, sxyz_ref, mask_ref, gidx_ref, wb_ref):
    x = xyz_ref[...]
    s = sxyz_ref[...]
    mask = mask_ref[...]

    x16 = x.astype(jnp.bfloat16)
    s16 = s.astype(jnp.bfloat16)
    a2 = jnp.sum(x * x, axis=1, keepdims=True)
    b2 = jnp.sum(s * s, axis=0, keepdims=True)
    cross = lax.dot_general(x16, s16, (((1,), (0,)), ((), ())),
                            preferred_element_type=jnp.float32)
    d = a2 + b2 - 2.0 * cross

    iota = lax.broadcasted_iota(jnp.int32, (TN, M), 1).astype(jnp.float32)
    inf = jnp.float32(jnp.inf)
    mf = jnp.float32(M)

    def extract(dcur):
        mval = jnp.min(dcur, axis=1, keepdims=True)
        aidx = jnp.min(jnp.where(dcur == mval, iota, mf), axis=1, keepdims=True)
        return mval, aidx

    m1, a1 = extract(d)
    d = jnp.where(iota == a1, inf, d)
    m2, a2 = extract(d)
    d = jnp.where(iota == a2, inf, d)
    m3, a3 = extract(d)

    def weight(mv):
        v = jnp.maximum(mv, jnp.float32(1e-10))
        return 1.0 / (v * v + jnp.float32(1e-10))

    w1 = weight(m1)
    w2 = weight(m2)
    w3 = weight(m3)
    wsum = w1 + w2 + w3
    w1 = w1 / wsum * mask
    w2 = w2 / wsum * mask
    w3 = w3 / wsum * mask

    gidx_ref[...] = jnp.concatenate([a1, a2, a3], axis=1).astype(jnp.int32)
    wb_ref[...] = jnp.concatenate([
        jnp.broadcast_to(w1, (TN, LANES)),
        jnp.broadcast_to(w2, (TN, LANES)),
        jnp.broadcast_to(w3, (TN, LANES)),
    ], axis=1)


def _tc_topk(xyz_b, sxyz_t_b, mask_b):
    return pl.pallas_call(
        _topk_body,
        grid=(NT,),
        in_specs=[
            pl.BlockSpec((TN, 3), lambda t: (t, 0)),
            pl.BlockSpec((3, M), lambda t: (0, 0)),
            pl.BlockSpec((TN, 1), lambda t: (t, 0)),
        ],
        out_specs=[
            pl.BlockSpec((TN, KNN), lambda t: (t, 0)),
            pl.BlockSpec((TN, KNN * LANES), lambda t: (t, 0)),
        ],
        out_shape=[
            jax.ShapeDtypeStruct((N, KNN), jnp.int32),
            jax.ShapeDtypeStruct((N, KNN * LANES), jnp.float32),
        ],
    )(xyz_b, sxyz_t_b, mask_b)


def _sc_interp(table, gidx, wb):
    mesh = plsc.VectorSubcoreMesh(core_axis_name="c", subcore_axis_name="s")

    @functools.partial(
        pl.kernel,
        out_type=jax.ShapeDtypeStruct((N, D), jnp.float32),
        mesh=mesh,
        scratch_types=[
            pltpu.VMEM((KNN * C,), jnp.int32),
            pltpu.VMEM((KNN * C, LANES), jnp.float32),
            pltpu.VMEM((KNN * C, D), jnp.float32),
            pltpu.VMEM((C, D), jnp.float32),
            pltpu.SemaphoreType.DMA,
        ],
    )
    def sck(table_hbm, gidx_hbm, wb_hbm, out_hbm, idx_v, w_v, rows_v, out_v, sem):
        wid = lax.axis_index("s") * NC + lax.axis_index("c")

        @pl.loop(0, NCHUNK)
        def _(ch):
            base = wid * PW + ch * C
            ibase = base * KNN
            pltpu.sync_copy(gidx_hbm.at[pl.ds(ibase, KNN * C)], idx_v)
            pltpu.async_copy(table_hbm.at[idx_v], rows_v, sem).wait()
            pltpu.sync_copy(wb_hbm.at[pl.ds(ibase, KNN * C)], w_v)

            @pl.loop(0, C)
            def _(r):
                w0 = w_v[KNN * r]
                w1 = w_v[KNN * r + 1]
                w2 = w_v[KNN * r + 2]
                for c in range(D // LANES):
                    sl = pl.ds(c * LANES, LANES)
                    out_v[r, sl] = (rows_v[KNN * r, sl] * w0 +
                                    rows_v[KNN * r + 1, sl] * w1 +
                                    rows_v[KNN * r + 2, sl] * w2)

            pltpu.sync_copy(out_v, out_hbm.at[pl.ds(base, C)])

    return sck(table, gidx, wb)


@jax.jit
def kernel(xyz, sampled_xyz, features, sampled_features, masks):
    del features
    sxyz_t = sampled_xyz.transpose(0, 2, 1)
    mask_f = masks.astype(jnp.float32).reshape(B, N, 1)
    outs = []
    for b in range(B):
        gidx, wb = _tc_topk(xyz[b], sxyz_t[b], mask_f[b])
        out = _sc_interp(sampled_features[b],
                         gidx.reshape(N * KNN),
                         wb.reshape(N * KNN, LANES))
        outs.append(out)
    return jnp.stack(outs, axis=0)


# speedup vs baseline: 17.6348x; 1.0420x over previous
import functools

import jax
import jax.numpy as jnp
from jax import lax
from jax.experimental import pallas as pl
from jax.experimental.pallas import tpu as pltpu
from jax.experimental.pallas import tpu_sc as plsc

KNN = 3
B = 4
N = 16384
M = 4096
D = 256

TN = 256
NT = N // TN

NC = 2
NS = 16
LANES = 16
NW = NC * NS
PW = N // NW
C = 32
NCHUNK = PW // C


def _topk_body(xyz_ref, sxyz_ref, mask_ref, gidx_ref, wb_ref):
    x = xyz_ref[...]
    s = sxyz_ref[...]
    mask = mask_ref[...]

    x16 = x.astype(jnp.bfloat16)
    s16 = s.astype(jnp.bfloat16)
    a2 = jnp.sum(x * x, axis=1, keepdims=True)
    b2 = jnp.sum(s * s, axis=0, keepdims=True)
    cross = lax.dot_general(x16, s16, (((1,), (0,)), ((), ())),
                            preferred_element_type=jnp.float32)
    d = a2 + b2 - 2.0 * cross

    iota = lax.broadcasted_iota(jnp.int32, (TN, M), 1).astype(jnp.float32)
    inf = jnp.float32(jnp.inf)
    mf = jnp.float32(M)

    def extract(dcur):
        mval = jnp.min(dcur, axis=1, keepdims=True)
        aidx = jnp.min(jnp.where(dcur == mval, iota, mf), axis=1, keepdims=True)
        return mval, aidx

    m1, a1 = extract(d)
    d = jnp.where(iota == a1, inf, d)
    m2, a2 = extract(d)
    d = jnp.where(iota == a2, inf, d)
    m3, a3 = extract(d)

    def weight(mv):
        v = jnp.maximum(mv, jnp.float32(1e-10))
        return 1.0 / (v * v + jnp.float32(1e-10))

    w1 = weight(m1)
    w2 = weight(m2)
    w3 = weight(m3)
    wsum = w1 + w2 + w3
    w1 = w1 / wsum * mask
    w2 = w2 / wsum * mask
    w3 = w3 / wsum * mask

    gidx_ref[...] = jnp.concatenate([a1, a2, a3], axis=1).astype(jnp.int32)
    wb_ref[...] = jnp.concatenate([
        jnp.broadcast_to(w1, (TN, LANES)),
        jnp.broadcast_to(w2, (TN, LANES)),
        jnp.broadcast_to(w3, (TN, LANES)),
    ], axis=1)


def _tc_topk(xyz_b, sxyz_t_b, mask_b):
    return pl.pallas_call(
        _topk_body,
        grid=(NT,),
        in_specs=[
            pl.BlockSpec((TN, 3), lambda t: (t, 0)),
            pl.BlockSpec((3, M), lambda t: (0, 0)),
            pl.BlockSpec((TN, 1), lambda t: (t, 0)),
        ],
        out_specs=[
            pl.BlockSpec((TN, KNN), lambda t: (t, 0)),
            pl.BlockSpec((TN, KNN * LANES), lambda t: (t, 0)),
        ],
        out_shape=[
            jax.ShapeDtypeStruct((N, KNN), jnp.int32),
            jax.ShapeDtypeStruct((N, KNN * LANES), jnp.float32),
        ],
    )(xyz_b, sxyz_t_b, mask_b)


def _sc_interp(table, gidx, wb):
    mesh = plsc.VectorSubcoreMesh(core_axis_name="c", subcore_axis_name="s")
    G = KNN * C

    @functools.partial(
        pl.kernel,
        out_type=jax.ShapeDtypeStruct((N, D), jnp.float32),
        mesh=mesh,
        scratch_types=[
            pltpu.VMEM((G,), jnp.int32),
            pltpu.VMEM((G,), jnp.int32),
            pltpu.VMEM((G, LANES), jnp.float32),
            pltpu.VMEM((G, LANES), jnp.float32),
            pltpu.VMEM((G, D), jnp.float32),
            pltpu.VMEM((G, D), jnp.float32),
            pltpu.VMEM((C, D), jnp.float32),
            pltpu.SemaphoreType.DMA,
            pltpu.SemaphoreType.DMA,
            pltpu.SemaphoreType.DMA,
            pltpu.SemaphoreType.DMA,
            pltpu.SemaphoreType.DMA,
            pltpu.SemaphoreType.DMA,
        ],
    )
    def sck(table_hbm, gidx_hbm, wb_hbm, out_hbm,
            idx0, idx1, w0, w1, rows0, rows1, out_v,
            si0, si1, sw0, sw1, sg0, sg1):
        wid = lax.axis_index("s") * NC + lax.axis_index("c")
        idx_v = (idx0, idx1)
        w_v = (w0, w1)
        rows_v = (rows0, rows1)
        si = (si0, si1)
        sw = (sw0, sw1)
        sg = (sg0, sg1)

        def start_idx(ch, slot):
            @pl.when(ch < NCHUNK)
            def _():
                ibase = (wid * PW + ch * C) * KNN
                pltpu.async_copy(gidx_hbm.at[pl.ds(ibase, G)], idx_v[slot],
                                 si[slot])

        def start_w(ch, slot):
            @pl.when(ch < NCHUNK)
            def _():
                ibase = (wid * PW + ch * C) * KNN
                pltpu.async_copy(wb_hbm.at[pl.ds(ibase, G)], w_v[slot],
                                 sw[slot])

        def wait_idx(slot):
            pltpu.make_async_copy(gidx_hbm.at[pl.ds(0, G)], idx_v[slot],
                                  si[slot]).wait()

        def start_gather(slot):
            pltpu.async_copy(table_hbm.at[idx_v[slot]], rows_v[slot], sg[slot])

        def wait_gather(slot):
            pltpu.make_async_copy(table_hbm.at[idx_v[slot]], rows_v[slot],
                                  sg[slot]).wait()
            pltpu.make_async_copy(wb_hbm.at[pl.ds(0, G)], w_v[slot],
                                  sw[slot]).wait()

        def compute(ch, slot):
            rv = rows_v[slot]
            wv = w_v[slot]

            @pl.loop(0, C)
            def _(r):
                a = wv[KNN * r]
                b = wv[KNN * r + 1]
                c_ = wv[KNN * r + 2]
                for c in range(D // LANES):
                    sl = pl.ds(c * LANES, LANES)
                    out_v[r, sl] = (rv[KNN * r, sl] * a +
                                    rv[KNN * r + 1, sl] * b +
                                    rv[KNN * r + 2, sl] * c_)

            pltpu.sync_copy(out_v, out_hbm.at[pl.ds(wid * PW + ch * C, C)])

        start_idx(0, 0)
        start_w(0, 0)
        wait_idx(0)
        start_gather(0)
        start_idx(1, 1)
        start_w(1, 1)

        @pl.loop(0, NCHUNK, step=2)
        def _(ch):
            wait_idx(1)
            start_gather(1)
            wait_gather(0)
            start_idx(ch + 2, 0)
            compute(ch, 0)
            start_w(ch + 2, 0)

            @pl.when(ch + 2 < NCHUNK)
            def _():
                wait_idx(0)
                start_gather(0)

            wait_gather(1)
            start_idx(ch + 3, 1)
            compute(ch + 1, 1)
            start_w(ch + 3, 1)

    return sck(table, gidx, wb)


@jax.jit
def kernel(xyz, sampled_xyz, features, sampled_features, masks):
    del features
    sxyz_t = sampled_xyz.transpose(0, 2, 1)
    mask_f = masks.astype(jnp.float32).reshape(B, N, 1)
    outs = []
    for b in range(B):
        gidx, wb = _tc_topk(xyz[b], sxyz_t[b], mask_f[b])
        out = _sc_interp(sampled_features[b],
                         gidx.reshape(N * KNN),
                         wb.reshape(N * KNN, LANES))
        outs.append(out)
    return jnp.stack(outs, axis=0)


# speedup vs baseline: 18.2652x; 1.0357x over previous
import functools

import jax
import jax.numpy as jnp
from jax import lax
from jax.experimental import pallas as pl
from jax.experimental.pallas import tpu as pltpu
from jax.experimental.pallas import tpu_sc as plsc

KNN = 3
B = 4
N = 16384
M = 4096
D = 256

TN = 512
NT = N // TN

NC = 2
NS = 16
LANES = 16
NW = NC * NS
PW = N // NW
C = 32
NCHUNK = PW // C


def _topk_body(xyz_ref, sxyz_ref, mask_ref, gidx_ref, wb_ref):
    x = xyz_ref[...]
    s = sxyz_ref[...]
    mask = mask_ref[...]

    x16 = x.astype(jnp.bfloat16)
    s16 = s.astype(jnp.bfloat16)
    a2 = jnp.sum(x * x, axis=1, keepdims=True)
    b2 = jnp.sum(s * s, axis=0, keepdims=True)
    cross = lax.dot_general(x16, s16, (((1,), (0,)), ((), ())),
                            preferred_element_type=jnp.float32)
    d = a2 + b2 - 2.0 * cross

    iota = lax.broadcasted_iota(jnp.int32, (TN, M), 1).astype(jnp.float32)
    inf = jnp.float32(jnp.inf)
    mf = jnp.float32(M)

    def extract(dcur):
        mval = jnp.min(dcur, axis=1, keepdims=True)
        aidx = jnp.min(jnp.where(dcur == mval, iota, mf), axis=1, keepdims=True)
        return mval, aidx

    m1, a1 = extract(d)
    d = jnp.where(iota == a1, inf, d)
    m2, a2 = extract(d)
    d = jnp.where(iota == a2, inf, d)
    m3, a3 = extract(d)

    def weight(mv):
        v = jnp.maximum(mv, jnp.float32(1e-10))
        return 1.0 / (v * v + jnp.float32(1e-10))

    w1 = weight(m1)
    w2 = weight(m2)
    w3 = weight(m3)
    wsum = w1 + w2 + w3
    w1 = w1 / wsum * mask
    w2 = w2 / wsum * mask
    w3 = w3 / wsum * mask

    gidx_ref[...] = jnp.concatenate([a1, a2, a3], axis=1).astype(jnp.int32)
    wb_ref[...] = jnp.concatenate([
        jnp.broadcast_to(w1, (TN, LANES)),
        jnp.broadcast_to(w2, (TN, LANES)),
        jnp.broadcast_to(w3, (TN, LANES)),
    ], axis=1)


def _tc_topk(xyz_b, sxyz_t_b, mask_b):
    return pl.pallas_call(
        _topk_body,
        grid=(NT,),
        in_specs=[
            pl.BlockSpec((TN, 3), lambda t: (t, 0)),
            pl.BlockSpec((3, M), lambda t: (0, 0)),
            pl.BlockSpec((TN, 1), lambda t: (t, 0)),
        ],
        out_specs=[
            pl.BlockSpec((TN, KNN), lambda t: (t, 0)),
            pl.BlockSpec((TN, KNN * LANES), lambda t: (t, 0)),
        ],
        out_shape=[
            jax.ShapeDtypeStruct((N, KNN), jnp.int32),
            jax.ShapeDtypeStruct((N, KNN * LANES), jnp.float32),
        ],
    )(xyz_b, sxyz_t_b, mask_b)


def _sc_interp(table, gidx, wb):
    mesh = plsc.VectorSubcoreMesh(core_axis_name="c", subcore_axis_name="s")
    G = KNN * C

    @functools.partial(
        pl.kernel,
        out_type=jax.ShapeDtypeStruct((N, D), jnp.float32),
        mesh=mesh,
        scratch_types=[
            pltpu.VMEM((G,), jnp.int32),
            pltpu.VMEM((G,), jnp.int32),
            pltpu.VMEM((G, LANES), jnp.float32),
            pltpu.VMEM((G, LANES), jnp.float32),
            pltpu.VMEM((G, D), jnp.float32),
            pltpu.VMEM((G, D), jnp.float32),
            pltpu.VMEM((C, D), jnp.float32),
            pltpu.SemaphoreType.DMA,
            pltpu.SemaphoreType.DMA,
            pltpu.SemaphoreType.DMA,
            pltpu.SemaphoreType.DMA,
            pltpu.SemaphoreType.DMA,
            pltpu.SemaphoreType.DMA,
        ],
    )
    def sck(table_hbm, gidx_hbm, wb_hbm, out_hbm,
            idx0, idx1, w0, w1, rows0, rows1, out_v,
            si0, si1, sw0, sw1, sg0, sg1):
        wid = lax.axis_index("s") * NC + lax.axis_index("c")
        idx_v = (idx0, idx1)
        w_v = (w0, w1)
        rows_v = (rows0, rows1)
        si = (si0, si1)
        sw = (sw0, sw1)
        sg = (sg0, sg1)

        def start_idx(ch, slot):
            @pl.when(ch < NCHUNK)
            def _():
                ibase = (wid * PW + ch * C) * KNN
                pltpu.async_copy(gidx_hbm.at[pl.ds(ibase, G)], idx_v[slot],
                                 si[slot])

        def start_w(ch, slot):
            @pl.when(ch < NCHUNK)
            def _():
                ibase = (wid * PW + ch * C) * KNN
                pltpu.async_copy(wb_hbm.at[pl.ds(ibase, G)], w_v[slot],
                                 sw[slot])

        def wait_idx(slot):
            pltpu.make_async_copy(gidx_hbm.at[pl.ds(0, G)], idx_v[slot],
                                  si[slot]).wait()

        def start_gather(slot):
            pltpu.async_copy(table_hbm.at[idx_v[slot]], rows_v[slot], sg[slot])

        def wait_gather(slot):
            pltpu.make_async_copy(table_hbm.at[idx_v[slot]], rows_v[slot],
                                  sg[slot]).wait()
            pltpu.make_async_copy(wb_hbm.at[pl.ds(0, G)], w_v[slot],
                                  sw[slot]).wait()

        def compute(ch, slot):
            rv = rows_v[slot]
            wv = w_v[slot]

            @pl.loop(0, C)
            def _(r):
                a = wv[KNN * r]
                b = wv[KNN * r + 1]
                c_ = wv[KNN * r + 2]
                for c in range(D // LANES):
                    sl = pl.ds(c * LANES, LANES)
                    out_v[r, sl] = (rv[KNN * r, sl] * a +
                                    rv[KNN * r + 1, sl] * b +
                                    rv[KNN * r + 2, sl] * c_)

            pltpu.sync_copy(out_v, out_hbm.at[pl.ds(wid * PW + ch * C, C)])

        start_idx(0, 0)
        start_w(0, 0)
        wait_idx(0)
        start_gather(0)
        start_idx(1, 1)
        start_w(1, 1)

        @pl.loop(0, NCHUNK, step=2)
        def _(ch):
            wait_idx(1)
            start_gather(1)
            wait_gather(0)
            start_idx(ch + 2, 0)
            compute(ch, 0)
            start_w(ch + 2, 0)

            @pl.when(ch + 2 < NCHUNK)
            def _():
                wait_idx(0)
                start_gather(0)

            wait_gather(1)
            start_idx(ch + 3, 1)
            compute(ch + 1, 1)
            start_w(ch + 3, 1)

    return sck(table, gidx, wb)


@jax.jit
def kernel(xyz, sampled_xyz, features, sampled_features, masks):
    del features
    sxyz_t = sampled_xyz.transpose(0, 2, 1)
    mask_f = masks.astype(jnp.float32).reshape(B, N, 1)
    outs = []
    for b in range(B):
        gidx, wb = _tc_topk(xyz[b], sxyz_t[b], mask_f[b])
        out = _sc_interp(sampled_features[b],
                         gidx.reshape(N * KNN),
                         wb.reshape(N * KNN, LANES))
        outs.append(out)
    return jnp.stack(outs, axis=0)
